# baseline (speedup 1.0000x reference)
import jax
import jax.numpy as jnp
import numpy as np
from jax import lax
from jax.experimental import pallas as pl
from jax.experimental.pallas import tpu as pltpu
from jax.experimental.pallas import tpu_sc as plsc

GRAPH_NUM = 3
D = 128
NSAMP = 8192
HALF = NSAMP // 2
LEAKY = 0.2

_NC = 2
_NS = 16
_L = 16
_NW = _NC * _NS
_BPW = NSAMP // _NW
_NBUF = 3


def _leaky(x):
    return jnp.where(x > 0, x, LEAKY * x)


def _sc_gather_all(fu, uvf, fi, ivf, su, si, n_users, n_items):
    def body(fu_hbm, uvf_hbm, fi_hbm, ivf_hbm, su_hbm, si_hbm,
             fug, uvg, fig, ivg,
             isu0, isu1, isu2, isi0, isi1, isi2, iuv1, iuv2, iiv1, iiv2,
             rows0, rows1, bf0, bf1,
             isem, gsem0, gsem1, ssem0, ssem1):
        wid = lax.axis_index("s") * _NC + lax.axis_index("c")
        base = wid * _BPW
        base2 = wid * (_BPW // 2)
        rows = (rows0, rows1)
        bf = (bf0, bf1)
        gsems = (gsem0, gsem1)
        ssems = (ssem0, ssem1)
        isu = (isu0, isu1, isu2)
        isi = (isi0, isi1, isi2)

        loads = []
        for g in range(GRAPH_NUM):
            loads.append(pltpu.async_copy(
                su_hbm.at[pl.ds(g * NSAMP + base, _BPW)], isu[g], isem))
            loads.append(pltpu.async_copy(
                si_hbm.at[pl.ds(g * NSAMP + base, _BPW)], isi[g], isem))
        for c in loads:
            c.wait()

        for dst, srcv, off in ((iuv1, isu1, n_users), (iuv2, isu2, 2 * n_users),
                               (iiv1, isi1, n_items), (iiv2, isi2, 2 * n_items)):
            for k in range(_BPW // _L):
                sl = pl.ds(k * _L, _L)
                dst[sl] = srcv[sl] + off

        uv_idx = (isu0, iuv1, iuv2)
        iv_idx = (isi0, iiv1, iiv2)
        rounds = []
        for g in range(GRAPH_NUM):
            ob2 = g * (NSAMP // 2) + base2
            rounds.append((fu_hbm, isu[g], fug, ob2))
            rounds.append((uvf_hbm, uv_idx[g], uvg, ob2))
            rounds.append((fi_hbm, isi[g], fig, ob2))
            rounds.append((ivf_hbm, iv_idx[g], ivg, ob2))

        nr = len(rounds)
        gathers = [None] * nr
        scatters = [None] * nr

        def start_gather(r):
            tab, iref, _, _ = rounds[r]
            gathers[r] = pltpu.async_copy(tab.at[iref], rows[r % 2],
                                          gsems[r % 2])

        def convert(b):
            src, dst = rows[b], bf[b]

            def step(i, carry):
                for j in range(8):
                    r0 = i * 8 + j
                    dr = i * 4 + j // 2
                    cb = (j % 2) * (D // 2)
                    for q in range(D // (2 * _L)):
                        a = src[r0, pl.ds(q * 2 * _L, _L)]
                        c = src[r0, pl.ds(q * 2 * _L + _L, _L)]
                        ua = lax.bitcast_convert_type(a, jnp.int32) + 0x8000
                        uc = lax.bitcast_convert_type(c, jnp.int32) + 0x8000
                        dst[dr, pl.ds(cb + q * _L, _L)] = (
                            lax.shift_right_logical(ua, 16)
                            | (uc & jnp.int32(-65536)))
                return carry

            lax.fori_loop(0, _BPW // 8, step, 0)

        start_gather(0)
        start_gather(1)
        for r in range(nr):
            b = r % 2
            if r >= 2:
                scatters[r - 2].wait()
            gathers[r].wait()
            convert(b)
            if r + 2 < nr:
                start_gather(r + 2)
            _, _, out_ref, ob2 = rounds[r]
            scatters[r] = pltpu.async_copy(
                bf[b], out_ref.at[pl.ds(ob2, _BPW // 2)], ssems[b])
        scatters[nr - 2].wait()
        scatters[nr - 1].wait()

    out = jax.ShapeDtypeStruct((GRAPH_NUM * NSAMP // 2, D), jnp.int32)
    kern = pl.kernel(
        body,
        out_type=[out, out, out, out],
        mesh=plsc.VectorSubcoreMesh(core_axis_name="c", subcore_axis_name="s"),
        scratch_types=[
            pltpu.VMEM((_BPW,), jnp.int32),
            pltpu.VMEM((_BPW,), jnp.int32),
            pltpu.VMEM((_BPW,), jnp.int32),
            pltpu.VMEM((_BPW,), jnp.int32),
            pltpu.VMEM((_BPW,), jnp.int32),
            pltpu.VMEM((_BPW,), jnp.int32),
            pltpu.VMEM((_BPW,), jnp.int32),
            pltpu.VMEM((_BPW,), jnp.int32),
            pltpu.VMEM((_BPW,), jnp.int32),
            pltpu.VMEM((_BPW,), jnp.int32),
            pltpu.VMEM((_BPW, D), jnp.float32),
            pltpu.VMEM((_BPW, D), jnp.float32),
            pltpu.VMEM((_BPW // 2, D), jnp.int32),
            pltpu.VMEM((_BPW // 2, D), jnp.int32),
            pltpu.SemaphoreType.DMA,
            pltpu.SemaphoreType.DMA,
            pltpu.SemaphoreType.DMA,
            pltpu.SemaphoreType.DMA,
            pltpu.SemaphoreType.DMA,
        ],
    )
    return kern(fu, uvf, fi, ivf, su, si)


def _tc_body(fu_p, fu_n, uv_p, uv_n, fi_p, fi_n, iv_p, iv_n,
             w1, b1, w2, b2, out):
    @pl.when((pl.program_id(0) == 0) & (pl.program_id(1) == 0))
    def _():
        out[...] = jnp.zeros_like(out)

    W1 = w1[...]
    b1v = b1[...]
    w2v = w2[...]
    b2s = b2[0, 0]

    def weight(fu, uv):
        h = (jnp.dot(fu * uv, W1[:D], preferred_element_type=jnp.float32)
             + jnp.dot(fu, W1[D:2 * D], preferred_element_type=jnp.float32)
             + jnp.dot(uv, W1[2 * D:], preferred_element_type=jnp.float32)
             + b1v)
        h = _leaky(h)
        z = jnp.sum(h * w2v, axis=-1) + b2s
        return 1.0 / (1.0 + jnp.exp(-z))

    f32 = jnp.float32
    fu_pv, uv_pv = fu_p[...].astype(f32), uv_p[...].astype(f32)
    fu_nv, uv_nv = fu_n[...].astype(f32), uv_n[...].astype(f32)
    wpos = weight(fu_pv, uv_pv)
    wneg = weight(fu_nv, uv_nv)
    spos = jnp.sum(_leaky(fu_pv * fi_p[...].astype(f32)), axis=-1)
    sneg = jnp.sum(_leaky(fu_nv * fi_n[...].astype(f32)), axis=-1)
    ppos = jnp.sum(_leaky(uv_pv * iv_p[...].astype(f32)), axis=-1)
    pneg = jnp.sum(_leaky(uv_nv * iv_n[...].astype(f32)), axis=-1)
    s = wpos * spos - wneg * sneg
    l = jnp.sum(jnp.maximum(0.0, 1.0 - s * (ppos - pneg)))
    out[...] = out[...] + l


def _tc_loss(fug, uvg, fig, ivg, w1, b1r, w2r, b2r):
    BP = 1024
    nbj = HALF // BP
    nbg = NSAMP // BP

    rs_p = pl.BlockSpec((BP, D), lambda i, j: (i * nbg + j, 0))
    rs_n = pl.BlockSpec((BP, D), lambda i, j: (i * nbg + nbj + j, 0))

    def full(shape):
        return pl.BlockSpec(shape, lambda i, j: (0, 0))

    out = pl.pallas_call(
        _tc_body,
        grid=(GRAPH_NUM, nbj),
        in_specs=[rs_p, rs_n, rs_p, rs_n, rs_p, rs_n, rs_p, rs_n,
                  full((3 * D, D)), full((1, D)), full((1, D)), full((1, 1))],
        out_specs=pl.BlockSpec((1, 1), lambda i, j: (0, 0)),
        out_shape=jax.ShapeDtypeStruct((1, 1), jnp.float32),
    )(fug, fug, uvg, uvg, fig, fig, ivg, ivg, w1, b1r, w2r, b2r)
    return out[0, 0]


def kernel(final_user_vector, user_vector, final_item_vector, item_vector,
           suids0, suids1, suids2, siids0, siids1, siids2, W1, b1, W2, b2):
    n_users = final_user_vector.shape[0]
    n_items = final_item_vector.shape[0]
    su = jnp.concatenate([suids0, suids1, suids2]).astype(jnp.int32)
    si = jnp.concatenate([siids0, siids1, siids2]).astype(jnp.int32)
    uvf = user_vector.reshape(GRAPH_NUM * n_users, D)
    ivf = item_vector.reshape(GRAPH_NUM * n_items, D)
    gathered = _sc_gather_all(
        final_user_vector, uvf, final_item_vector, ivf, su, si,
        n_users, n_items)
    fug, uvg, fig, ivg = (
        jax.lax.bitcast_convert_type(x, jnp.bfloat16).reshape(
            GRAPH_NUM * NSAMP, D)
        for x in gathered)
    pi32 = np.array([v for i in range(_L) for v in (i, _L + i)])
    perm_d = np.concatenate([q * 2 * _L + pi32 for q in range(D // (2 * _L))])
    perm_rows = np.concatenate([t * D + perm_d for t in range(3)])
    w1p = jnp.take(W1, jnp.asarray(perm_rows), axis=0)
    return _tc_loss(fug, uvg, fig, ivg, w1p,
                    b1.reshape(1, D), W2.reshape(1, D), b2.reshape(1, 1))


# speedup vs baseline: 42.2811x; 42.2811x over previous
import jax
import jax.numpy as jnp
from jax import lax
from jax.experimental import pallas as pl
from jax.experimental.pallas import tpu as pltpu
from jax.experimental.pallas import tpu_sc as plsc

GRAPH_NUM = 3
D = 128
NSAMP = 8192
HALF = NSAMP // 2
LEAKY = 0.2

_NC = 2
_NS = 16
_L = 16
_NW = _NC * _NS
_BPW = NSAMP // _NW


def _leaky(x):
    return jnp.where(x > 0, x, LEAKY * x)


def _sc_gather_all(fu, uvf, fi, ivf, su, si, n_users, n_items):
    def body(fu_hbm, uvf_hbm, fi_hbm, ivf_hbm, su_hbm, si_hbm,
             fug, uvg, fig, ivg,
             isu0, isu1, isu2, isi0, isi1, isi2, iuv1, iuv2, iiv1, iiv2,
             rows0, rows1, bfb,
             isem, gsem0, gsem1, ssem):
        wid = lax.axis_index("s") * _NC + lax.axis_index("c")
        base = wid * _BPW
        rows = (rows0, rows1)
        gsems = (gsem0, gsem1)
        isu = (isu0, isu1, isu2)
        isi = (isi0, isi1, isi2)

        loads = []
        for g in range(GRAPH_NUM):
            loads.append(pltpu.async_copy(
                su_hbm.at[pl.ds(g * NSAMP + base, _BPW)], isu[g], isem))
            loads.append(pltpu.async_copy(
                si_hbm.at[pl.ds(g * NSAMP + base, _BPW)], isi[g], isem))
        for c in loads:
            c.wait()

        for dst, srcv, off in ((iuv1, isu1, n_users), (iuv2, isu2, 2 * n_users),
                               (iiv1, isi1, n_items), (iiv2, isi2, 2 * n_items)):
            for k in range(_BPW // _L):
                sl = pl.ds(k * _L, _L)
                dst[sl] = srcv[sl] + off

        uv_idx = (isu0, iuv1, iuv2)
        iv_idx = (isi0, iiv1, iiv2)
        rounds = []
        for g in range(GRAPH_NUM):
            ob = g * NSAMP + base
            rounds.append((fu_hbm, isu[g], fug, ob))
            rounds.append((uvf_hbm, uv_idx[g], uvg, ob))
            rounds.append((fi_hbm, isi[g], fig, ob))
            rounds.append((ivf_hbm, iv_idx[g], ivg, ob))

        nr = len(rounds)
        gathers = [None] * nr
        scatters = [None] * nr

        def start_gather(r):
            tab, iref, _, _ = rounds[r]
            gathers[r] = pltpu.async_copy(tab.at[iref], rows[r % 2],
                                          gsems[r % 2])

        def convert(b):
            src = rows[b]

            def step(i, carry):
                for j in range(8):
                    r0 = i * 8 + j
                    for q in range(D // (2 * _L)):
                        a = src[r0, pl.ds(q * 2 * _L, _L)]
                        c = src[r0, pl.ds(q * 2 * _L + _L, _L)]
                        ua = lax.bitcast_convert_type(a, jnp.int32) + 0x8000
                        uc = lax.bitcast_convert_type(c, jnp.int32) + 0x8000
                        bfb[r0, pl.ds(q * _L, _L)] = (
                            lax.shift_right_logical(ua, 16)
                            | (uc & jnp.int32(-65536)))
                return carry

            lax.fori_loop(0, _BPW // 8, step, 0)

        start_gather(0)
        start_gather(1)
        for r in range(nr):
            gathers[r].wait()
            if r >= 1:
                scatters[r - 1].wait()
            convert(r % 2)
            if r + 2 < nr:
                start_gather(r + 2)
            _, _, out_ref, ob = rounds[r]
            scatters[r] = pltpu.async_copy(
                bfb, out_ref.at[pl.ds(ob, _BPW)], ssem)
        scatters[nr - 1].wait()

    out = jax.ShapeDtypeStruct((GRAPH_NUM * NSAMP, D // 2), jnp.int32)
    kern = pl.kernel(
        body,
        out_type=[out, out, out, out],
        mesh=plsc.VectorSubcoreMesh(core_axis_name="c", subcore_axis_name="s"),
        scratch_types=[
            pltpu.VMEM((_BPW,), jnp.int32),
            pltpu.VMEM((_BPW,), jnp.int32),
            pltpu.VMEM((_BPW,), jnp.int32),
            pltpu.VMEM((_BPW,), jnp.int32),
            pltpu.VMEM((_BPW,), jnp.int32),
            pltpu.VMEM((_BPW,), jnp.int32),
            pltpu.VMEM((_BPW,), jnp.int32),
            pltpu.VMEM((_BPW,), jnp.int32),
            pltpu.VMEM((_BPW,), jnp.int32),
            pltpu.VMEM((_BPW,), jnp.int32),
            pltpu.VMEM((_BPW, D), jnp.float32),
            pltpu.VMEM((_BPW, D), jnp.float32),
            pltpu.VMEM((_BPW, D // 2), jnp.int32),
            pltpu.SemaphoreType.DMA,
            pltpu.SemaphoreType.DMA,
            pltpu.SemaphoreType.DMA,
            pltpu.SemaphoreType.DMA,
        ],
    )
    return kern(fu, uvf, fi, ivf, su, si)


def _unpack2(x):
    a = lax.bitcast_convert_type(lax.shift_left(x, 16), jnp.float32)
    c = lax.bitcast_convert_type(x & jnp.int32(-65536), jnp.float32)
    return a, c


def _tc_body(fu_p, fu_n, uv_p, uv_n, fi_p, fi_n, iv_p, iv_n,
             w1a, w1c, b1, w2, b2, out):
    @pl.when((pl.program_id(0) == 0) & (pl.program_id(1) == 0))
    def _():
        out[...] = jnp.zeros_like(out)

    W1a = w1a[...]
    W1c = w1c[...]
    b1v = b1[...]
    w2v = w2[...]
    b2s = b2[0, 0]
    dh = D // 2

    afu_p, cfu_p = _unpack2(fu_p[...])
    auv_p, cuv_p = _unpack2(uv_p[...])
    afu_n, cfu_n = _unpack2(fu_n[...])
    auv_n, cuv_n = _unpack2(uv_n[...])
    afi_p, cfi_p = _unpack2(fi_p[...])
    afi_n, cfi_n = _unpack2(fi_n[...])
    aiv_p, civ_p = _unpack2(iv_p[...])
    aiv_n, civ_n = _unpack2(iv_n[...])

    def dot(x, w):
        return jnp.dot(x, w, preferred_element_type=jnp.float32)

    def weight(afu, cfu, auv, cuv):
        h = (dot(afu * auv, W1a[:dh]) + dot(cfu * cuv, W1c[:dh])
             + dot(afu, W1a[dh:2 * dh]) + dot(cfu, W1c[dh:2 * dh])
             + dot(auv, W1a[2 * dh:]) + dot(cuv, W1c[2 * dh:])
             + b1v)
        h = _leaky(h)
        z = jnp.sum(h * w2v, axis=-1) + b2s
        return 1.0 / (1.0 + jnp.exp(-z))

    wpos = weight(afu_p, cfu_p, auv_p, cuv_p)
    wneg = weight(afu_n, cfu_n, auv_n, cuv_n)
    spos = (jnp.sum(_leaky(afu_p * afi_p), axis=-1)
            + jnp.sum(_leaky(cfu_p * cfi_p), axis=-1))
    sneg = (jnp.sum(_leaky(afu_n * afi_n), axis=-1)
            + jnp.sum(_leaky(cfu_n * cfi_n), axis=-1))
    ppos = (jnp.sum(_leaky(auv_p * aiv_p), axis=-1)
            + jnp.sum(_leaky(cuv_p * civ_p), axis=-1))
    pneg = (jnp.sum(_leaky(auv_n * aiv_n), axis=-1)
            + jnp.sum(_leaky(cuv_n * civ_n), axis=-1))
    s = wpos * spos - wneg * sneg
    l = jnp.sum(jnp.maximum(0.0, 1.0 - s * (ppos - pneg)))
    out[...] = out[...] + l


def _tc_loss(fug, uvg, fig, ivg, w1a, w1c, b1r, w2r, b2r):
    BP = 1024
    nbj = HALF // BP
    nbg = NSAMP // BP

    rs_p = pl.BlockSpec((BP, D // 2), lambda i, j: (i * nbg + j, 0))
    rs_n = pl.BlockSpec((BP, D // 2), lambda i, j: (i * nbg + nbj + j, 0))

    def full(shape):
        return pl.BlockSpec(shape, lambda i, j: (0, 0))

    out = pl.pallas_call(
        _tc_body,
        grid=(GRAPH_NUM, nbj),
        in_specs=[rs_p, rs_n, rs_p, rs_n, rs_p, rs_n, rs_p, rs_n,
                  full((3 * D // 2, D)), full((3 * D // 2, D)),
                  full((1, D)), full((1, D)), full((1, 1))],
        out_specs=pl.BlockSpec((1, 1), lambda i, j: (0, 0)),
        out_shape=jax.ShapeDtypeStruct((1, 1), jnp.float32),
    )(fug, fug, uvg, uvg, fig, fig, ivg, ivg, w1a, w1c, b1r, w2r, b2r)
    return out[0, 0]


def kernel(final_user_vector, user_vector, final_item_vector, item_vector,
           suids0, suids1, suids2, siids0, siids1, siids2, W1, b1, W2, b2):
    n_users = final_user_vector.shape[0]
    n_items = final_item_vector.shape[0]
    su = jnp.concatenate([suids0, suids1, suids2]).astype(jnp.int32)
    si = jnp.concatenate([siids0, siids1, siids2]).astype(jnp.int32)
    uvf = user_vector.reshape(GRAPH_NUM * n_users, D)
    ivf = item_vector.reshape(GRAPH_NUM * n_items, D)
    fug, uvg, fig, ivg = _sc_gather_all(
        final_user_vector, uvf, final_item_vector, ivf, su, si,
        n_users, n_items)
    w1r = W1.reshape(3, D // (2 * _L), 2, _L, D)
    w1a = w1r[:, :, 0].reshape(3 * D // 2, D)
    w1c = w1r[:, :, 1].reshape(3 * D // 2, D)
    return _tc_loss(fug, uvg, fig, ivg, w1a, w1c,
                    b1.reshape(1, D), W2.reshape(1, D), b2.reshape(1, 1))


# speedup vs baseline: 42.9681x; 1.0162x over previous
import jax
import jax.numpy as jnp
from jax import lax
from jax.experimental import pallas as pl
from jax.experimental.pallas import tpu as pltpu
from jax.experimental.pallas import tpu_sc as plsc

GRAPH_NUM = 3
D = 128
NSAMP = 8192
HALF = NSAMP // 2
LEAKY = 0.2

_NC = 2
_NS = 16
_L = 16
_NW = _NC * _NS
_BPW = NSAMP // _NW


def _leaky(x):
    return jnp.where(x > 0, x, LEAKY * x)


def _sc_gather_all(fu, uvf, fi, ivf, su, si, n_users, n_items):
    def body(fu_hbm, uvf_hbm, fi_hbm, ivf_hbm, su_hbm, si_hbm,
             fug, uvg, fig, ivg,
             isu0, isu1, isu2, isi0, isi1, isi2, iuv1, iuv2, iiv1, iiv2,
             rows0, rows1, bfb,
             isem, gsem0, gsem1, ssem):
        wid = lax.axis_index("s") * _NC + lax.axis_index("c")
        base = wid * _BPW
        rows = (rows0, rows1)
        gsems = (gsem0, gsem1)
        isu = (isu0, isu1, isu2)
        isi = (isi0, isi1, isi2)

        loads = []
        for g in range(GRAPH_NUM):
            loads.append(pltpu.async_copy(
                su_hbm.at[pl.ds(g * NSAMP + base, _BPW)], isu[g], isem))
            loads.append(pltpu.async_copy(
                si_hbm.at[pl.ds(g * NSAMP + base, _BPW)], isi[g], isem))
        for c in loads:
            c.wait()

        for dst, srcv, off in ((iuv1, isu1, n_users), (iuv2, isu2, 2 * n_users),
                               (iiv1, isi1, n_items), (iiv2, isi2, 2 * n_items)):
            for k in range(_BPW // _L):
                sl = pl.ds(k * _L, _L)
                dst[sl] = srcv[sl] + off

        uv_idx = (isu0, iuv1, iuv2)
        iv_idx = (isi0, iiv1, iiv2)
        rounds = []
        for g in range(GRAPH_NUM):
            ob = g * NSAMP + base
            rounds.append((fu_hbm, isu[g], fug, ob))
            rounds.append((uvf_hbm, uv_idx[g], uvg, ob))
            rounds.append((fi_hbm, isi[g], fig, ob))
            rounds.append((ivf_hbm, iv_idx[g], ivg, ob))

        nr = len(rounds)
        gathers = [None] * nr
        scatters = [None] * nr

        def start_gather(r):
            tab, iref, _, _ = rounds[r]
            gathers[r] = pltpu.async_copy(tab.at[iref], rows[r % 2],
                                          gsems[r % 2])

        def convert(b):
            src = rows[b]

            def step(i, carry):
                for j in range(8):
                    r0 = i * 8 + j
                    for q in range(D // (2 * _L)):
                        a = src[r0, pl.ds(q * 2 * _L, _L)]
                        c = src[r0, pl.ds(q * 2 * _L + _L, _L)]
                        ua = lax.bitcast_convert_type(a, jnp.int32) + 0x8000
                        uc = lax.bitcast_convert_type(c, jnp.int32) + 0x8000
                        bfb[r0, pl.ds(q * _L, _L)] = (
                            lax.shift_right_logical(ua, 16)
                            | (uc & jnp.int32(-65536)))
                return carry

            lax.fori_loop(0, _BPW // 8, step, 0)

        start_gather(0)
        start_gather(1)
        for r in range(nr):
            gathers[r].wait()
            if r >= 1:
                scatters[r - 1].wait()
            convert(r % 2)
            if r + 2 < nr:
                start_gather(r + 2)
            _, _, out_ref, ob = rounds[r]
            scatters[r] = pltpu.async_copy(
                bfb, out_ref.at[pl.ds(ob, _BPW)], ssem)
        scatters[nr - 1].wait()

    out = jax.ShapeDtypeStruct((GRAPH_NUM * NSAMP, D // 2), jnp.int32)
    kern = pl.kernel(
        body,
        out_type=[out, out, out, out],
        mesh=plsc.VectorSubcoreMesh(core_axis_name="c", subcore_axis_name="s"),
        scratch_types=[
            pltpu.VMEM((_BPW,), jnp.int32),
            pltpu.VMEM((_BPW,), jnp.int32),
            pltpu.VMEM((_BPW,), jnp.int32),
            pltpu.VMEM((_BPW,), jnp.int32),
            pltpu.VMEM((_BPW,), jnp.int32),
            pltpu.VMEM((_BPW,), jnp.int32),
            pltpu.VMEM((_BPW,), jnp.int32),
            pltpu.VMEM((_BPW,), jnp.int32),
            pltpu.VMEM((_BPW,), jnp.int32),
            pltpu.VMEM((_BPW,), jnp.int32),
            pltpu.VMEM((_BPW, D), jnp.float32),
            pltpu.VMEM((_BPW, D), jnp.float32),
            pltpu.VMEM((_BPW, D // 2), jnp.int32),
            pltpu.SemaphoreType.DMA,
            pltpu.SemaphoreType.DMA,
            pltpu.SemaphoreType.DMA,
            pltpu.SemaphoreType.DMA,
        ],
    )
    return kern(fu, uvf, fi, ivf, su, si)


def _unpack(x):
    a = lax.bitcast_convert_type(lax.shift_left(x, 16), jnp.float32)
    c = lax.bitcast_convert_type(x & jnp.int32(-65536), jnp.float32)
    return jnp.concatenate([a, c], axis=-1)


def _tc_body(fu_p, fu_n, uv_p, uv_n, fi_p, fi_n, iv_p, iv_n,
             w1m, b1, w2, b2, out):
    @pl.when((pl.program_id(0) == 0) & (pl.program_id(1) == 0))
    def _():
        out[...] = jnp.zeros_like(out)

    W1m = w1m[...]
    b1v = b1[...]
    w2v = w2[...]
    b2s = b2[0, 0]

    fu_pv, uv_pv = _unpack(fu_p[...]), _unpack(uv_p[...])
    fu_nv, uv_nv = _unpack(fu_n[...]), _unpack(uv_n[...])

    def dot(x, w):
        return jnp.dot(x, w, preferred_element_type=jnp.float32)

    def weight(fu, uv):
        h = (dot(fu * uv, W1m[:D]) + dot(fu, W1m[D:2 * D])
             + dot(uv, W1m[2 * D:]) + b1v)
        h = _leaky(h)
        z = jnp.sum(h * w2v, axis=-1) + b2s
        return 1.0 / (1.0 + jnp.exp(-z))

    wpos = weight(fu_pv, uv_pv)
    wneg = weight(fu_nv, uv_nv)
    spos = jnp.sum(_leaky(fu_pv * _unpack(fi_p[...])), axis=-1)
    sneg = jnp.sum(_leaky(fu_nv * _unpack(fi_n[...])), axis=-1)
    ppos = jnp.sum(_leaky(uv_pv * _unpack(iv_p[...])), axis=-1)
    pneg = jnp.sum(_leaky(uv_nv * _unpack(iv_n[...])), axis=-1)
    s = wpos * spos - wneg * sneg
    l = jnp.sum(jnp.maximum(0.0, 1.0 - s * (ppos - pneg)))
    out[...] = out[...] + l


def _tc_loss(fug, uvg, fig, ivg, w1m, b1r, w2r, b2r):
    BP = 2048
    nbj = HALF // BP
    nbg = NSAMP // BP

    rs_p = pl.BlockSpec((BP, D // 2), lambda i, j: (i * nbg + j, 0))
    rs_n = pl.BlockSpec((BP, D // 2), lambda i, j: (i * nbg + nbj + j, 0))

    def full(shape):
        return pl.BlockSpec(shape, lambda i, j: (0, 0))

    out = pl.pallas_call(
        _tc_body,
        grid=(GRAPH_NUM, nbj),
        in_specs=[rs_p, rs_n, rs_p, rs_n, rs_p, rs_n, rs_p, rs_n,
                  full((3 * D, D)),
                  full((1, D)), full((1, D)), full((1, 1))],
        out_specs=pl.BlockSpec((1, 1), lambda i, j: (0, 0)),
        out_shape=jax.ShapeDtypeStruct((1, 1), jnp.float32),
    )(fug, fug, uvg, uvg, fig, fig, ivg, ivg, w1m, b1r, w2r, b2r)
    return out[0, 0]


def kernel(final_user_vector, user_vector, final_item_vector, item_vector,
           suids0, suids1, suids2, siids0, siids1, siids2, W1, b1, W2, b2):
    n_users = final_user_vector.shape[0]
    n_items = final_item_vector.shape[0]
    su = jnp.concatenate([suids0, suids1, suids2]).astype(jnp.int32)
    si = jnp.concatenate([siids0, siids1, siids2]).astype(jnp.int32)
    uvf = user_vector.reshape(GRAPH_NUM * n_users, D)
    ivf = item_vector.reshape(GRAPH_NUM * n_items, D)
    fug, uvg, fig, ivg = _sc_gather_all(
        final_user_vector, uvf, final_item_vector, ivf, su, si,
        n_users, n_items)
    w1r = W1.reshape(3, D // (2 * _L), 2, _L, D)
    w1m = w1r.transpose(0, 2, 1, 3, 4).reshape(3 * D, D)
    return _tc_loss(fug, uvg, fig, ivg, w1m,
                    b1.reshape(1, D), W2.reshape(1, D), b2.reshape(1, 1))


# speedup vs baseline: 51.5341x; 1.1994x over previous
import jax
import jax.numpy as jnp
from jax import lax
from jax.experimental import pallas as pl
from jax.experimental.pallas import tpu as pltpu
from jax.experimental.pallas import tpu_sc as plsc

GRAPH_NUM = 3
D = 128
NSAMP = 8192
HALF = NSAMP // 2
LEAKY = 0.2

_NC = 2
_NS = 16
_L = 16
_NW = _NC * _NS
_BPW = NSAMP // _NW
_NBUF = 3


def _leaky(x):
    return jnp.where(x > 0, x, LEAKY * x)


def _sc_gather_all(fu, uvf, fi, ivf, su, si, n_users, n_items):
    def body(fu_hbm, uvf_hbm, fi_hbm, ivf_hbm, su_hbm, si_hbm,
             fug, uvg, fig, ivg,
             isu0, isu1, isu2, isi0, isi1, isi2, iuv1, iuv2, iiv1, iiv2,
             rows0, rows1, rows2,
             isem, gsem0, gsem1, gsem2, ssem0, ssem1, ssem2):
        wid = lax.axis_index("s") * _NC + lax.axis_index("c")
        base = wid * _BPW
        rows = (rows0, rows1, rows2)
        gsems = (gsem0, gsem1, gsem2)
        ssems = (ssem0, ssem1, ssem2)
        isu = (isu0, isu1, isu2)
        isi = (isi0, isi1, isi2)

        loads = []
        for g in range(GRAPH_NUM):
            loads.append(pltpu.async_copy(
                su_hbm.at[pl.ds(g * NSAMP + base, _BPW)], isu[g], isem))
            loads.append(pltpu.async_copy(
                si_hbm.at[pl.ds(g * NSAMP + base, _BPW)], isi[g], isem))
        for c in loads:
            c.wait()

        for dst, srcv, off in ((iuv1, isu1, n_users), (iuv2, isu2, 2 * n_users),
                               (iiv1, isi1, n_items), (iiv2, isi2, 2 * n_items)):
            for k in range(_BPW // _L):
                sl = pl.ds(k * _L, _L)
                dst[sl] = srcv[sl] + off

        uv_idx = (isu0, iuv1, iuv2)
        iv_idx = (isi0, iiv1, iiv2)
        rounds = []
        for g in range(GRAPH_NUM):
            ob = g * NSAMP + base
            rounds.append((fu_hbm, isu[g], fug, ob))
            rounds.append((uvf_hbm, uv_idx[g], uvg, ob))
            rounds.append((fi_hbm, isi[g], fig, ob))
            rounds.append((ivf_hbm, iv_idx[g], ivg, ob))

        nr = len(rounds)
        gathers = [None] * nr
        scatters = [None] * nr

        def start_gather(r):
            tab, iref, _, _ = rounds[r]
            b = r % _NBUF
            gathers[r] = pltpu.async_copy(tab.at[iref], rows[b], gsems[b])

        def retire(r):
            _, _, out_ref, ob = rounds[r]
            b = r % _NBUF
            gathers[r].wait()
            scatters[r] = pltpu.async_copy(rows[b],
                                           out_ref.at[pl.ds(ob, _BPW)],
                                           ssems[b])

        for r in range(_NBUF):
            start_gather(r)
        for r in range(_NBUF, nr):
            retire(r - _NBUF)
            scatters[r - _NBUF].wait()
            start_gather(r)
        for r in range(nr - _NBUF, nr):
            retire(r)
        for r in range(nr - _NBUF, nr):
            scatters[r].wait()

    out = jax.ShapeDtypeStruct((GRAPH_NUM * NSAMP, D), jnp.float32)
    kern = pl.kernel(
        body,
        out_type=[out, out, out, out],
        mesh=plsc.VectorSubcoreMesh(core_axis_name="c", subcore_axis_name="s"),
        scratch_types=[
            pltpu.VMEM((_BPW,), jnp.int32),
            pltpu.VMEM((_BPW,), jnp.int32),
            pltpu.VMEM((_BPW,), jnp.int32),
            pltpu.VMEM((_BPW,), jnp.int32),
            pltpu.VMEM((_BPW,), jnp.int32),
            pltpu.VMEM((_BPW,), jnp.int32),
            pltpu.VMEM((_BPW,), jnp.int32),
            pltpu.VMEM((_BPW,), jnp.int32),
            pltpu.VMEM((_BPW,), jnp.int32),
            pltpu.VMEM((_BPW,), jnp.int32),
            pltpu.VMEM((_BPW, D), jnp.float32),
            pltpu.VMEM((_BPW, D), jnp.float32),
            pltpu.VMEM((_BPW, D), jnp.float32),
            pltpu.SemaphoreType.DMA,
            pltpu.SemaphoreType.DMA,
            pltpu.SemaphoreType.DMA,
            pltpu.SemaphoreType.DMA,
            pltpu.SemaphoreType.DMA,
            pltpu.SemaphoreType.DMA,
            pltpu.SemaphoreType.DMA,
        ],
    )
    return kern(fu, uvf, fi, ivf, su, si)


def _tc_body(fu_p, fu_n, uv_p, uv_n, fi_p, fi_n, iv_p, iv_n,
             w1, b1, w2, b2, out):
    @pl.when((pl.program_id(0) == 0) & (pl.program_id(1) == 0))
    def _():
        out[...] = jnp.zeros_like(out)

    W1 = w1[...]
    b1v = b1[...]
    w2v = w2[...]
    b2s = b2[0, 0]

    def weight(fu, uv):
        h = (jnp.dot(fu * uv, W1[:D], preferred_element_type=jnp.float32)
             + jnp.dot(fu, W1[D:2 * D], preferred_element_type=jnp.float32)
             + jnp.dot(uv, W1[2 * D:], preferred_element_type=jnp.float32)
             + b1v)
        h = _leaky(h)
        z = jnp.sum(h * w2v, axis=-1) + b2s
        return 1.0 / (1.0 + jnp.exp(-z))

    fu_pv, uv_pv = fu_p[...], uv_p[...]
    fu_nv, uv_nv = fu_n[...], uv_n[...]
    wpos = weight(fu_pv, uv_pv)
    wneg = weight(fu_nv, uv_nv)
    spos = jnp.sum(_leaky(fu_pv * fi_p[...]), axis=-1)
    sneg = jnp.sum(_leaky(fu_nv * fi_n[...]), axis=-1)
    ppos = jnp.sum(_leaky(uv_pv * iv_p[...]), axis=-1)
    pneg = jnp.sum(_leaky(uv_nv * iv_n[...]), axis=-1)
    s = wpos * spos - wneg * sneg
    l = jnp.sum(jnp.maximum(0.0, 1.0 - s * (ppos - pneg)))
    out[...] = out[...] + l


def _tc_loss(fug, uvg, fig, ivg, w1, b1r, w2r, b2r):
    BP = 2048
    nbj = HALF // BP
    nbg = NSAMP // BP

    rs_p = pl.BlockSpec((BP, D), lambda i, j: (i * nbg + j, 0))
    rs_n = pl.BlockSpec((BP, D), lambda i, j: (i * nbg + nbj + j, 0))

    def full(shape):
        return pl.BlockSpec(shape, lambda i, j: (0, 0))

    out = pl.pallas_call(
        _tc_body,
        grid=(GRAPH_NUM, nbj),
        in_specs=[rs_p, rs_n, rs_p, rs_n, rs_p, rs_n, rs_p, rs_n,
                  full((3 * D, D)), full((1, D)), full((1, D)), full((1, 1))],
        out_specs=pl.BlockSpec((1, 1), lambda i, j: (0, 0)),
        out_shape=jax.ShapeDtypeStruct((1, 1), jnp.float32),
    )(fug, fug, uvg, uvg, fig, fig, ivg, ivg, w1, b1r, w2r, b2r)
    return out[0, 0]


def kernel(final_user_vector, user_vector, final_item_vector, item_vector,
           suids0, suids1, suids2, siids0, siids1, siids2, W1, b1, W2, b2):
    n_users = final_user_vector.shape[0]
    n_items = final_item_vector.shape[0]
    su = jnp.concatenate([suids0, suids1, suids2]).astype(jnp.int32)
    si = jnp.concatenate([siids0, siids1, siids2]).astype(jnp.int32)
    uvf = user_vector.reshape(GRAPH_NUM * n_users, D)
    ivf = item_vector.reshape(GRAPH_NUM * n_items, D)
    fug, uvg, fig, ivg = _sc_gather_all(
        final_user_vector, uvf, final_item_vector, ivf, su, si,
        n_users, n_items)
    return _tc_loss(fug, uvg, fig, ivg, W1,
                    b1.reshape(1, D), W2.reshape(1, D), b2.reshape(1, 1))
